# baseline (speedup 1.0000x reference)
import functools

import jax
import jax.numpy as jnp
from jax import lax
from jax.experimental import pallas as pl
from jax.experimental.pallas import tpu as pltpu
from jax.experimental.pallas import tpu_sc as plsc

VOCAB = 1000000
EMBED_DIM = 64
NUM_CLASS = 1000
BATCH = 4096
SEQ = 200

NUM_CORES = 2
NUM_SUBCORES = 16
NUM_WORKERS = NUM_CORES * NUM_SUBCORES
B_PER_W = BATCH // NUM_WORKERS
ROW = 128
S0 = 128
S1 = SEQ - S0

NBUF = 2
UNROLL = 8


def _pool_body(ids_hbm, table_hbm, out_hbm, idx_v, gbuf, pooled_v, sems):
    wid = lax.axis_index("c") * NUM_SUBCORES + lax.axis_index("s")
    base = wid * B_PER_W
    pltpu.sync_copy(ids_hbm.at[pl.ds(base, B_PER_W), :], idx_v)

    inv_seq = jnp.float32(1.0 / SEQ)

    def start_gather(r, b):
        pltpu.async_copy(
            table_hbm.at[idx_v.at[r, pl.ds(0, S0)]],
            gbuf.at[b, pl.ds(0, S0)], sems.at[b])
        pltpu.async_copy(
            table_hbm.at[idx_v.at[r, pl.ds(S0, S1)]],
            gbuf.at[b, pl.ds(S0, S1)], sems.at[b])

    def wait_gather(b):
        pltpu.make_async_copy(
            table_hbm.at[idx_v.at[0, pl.ds(0, S0)]],
            gbuf.at[b, pl.ds(0, S0)], sems.at[b]).wait()
        pltpu.make_async_copy(
            table_hbm.at[idx_v.at[0, pl.ds(S0, S1)]],
            gbuf.at[b, pl.ds(S0, S1)], sems.at[b]).wait()

    def reduce_slot(r, b):
        def red_body(j, accs):
            accs = list(accs)
            for u in range(UNROLL):
                row = j * UNROLL + u
                for k in range(4):
                    a = u % 2 + 2 * k
                    accs[a] = accs[a] + gbuf[b, row, pl.ds(16 * k, 16)]
            return tuple(accs)

        zero = jnp.zeros((16,), jnp.float32)
        accs = lax.fori_loop(0, SEQ // UNROLL, red_body, (zero,) * 8)
        for k in range(4):
            pooled_v[r, pl.ds(16 * k, 16)] = (
                (accs[2 * k] + accs[2 * k + 1]) * inv_seq)

    for b in range(NBUF):
        start_gather(b, b)

    def outer(g, carry):
        for b in range(NBUF):
            r = g * NBUF + b
            wait_gather(b)
            reduce_slot(r, b)

            @pl.when(r + NBUF < B_PER_W)
            def _():
                start_gather(r + NBUF, b)
        return carry

    lax.fori_loop(0, B_PER_W // NBUF, outer, 0)
    pltpu.sync_copy(pooled_v, out_hbm.at[pl.ds(base, B_PER_W), :])


def _sc_pool(input_ids, table128):
    mesh = plsc.VectorSubcoreMesh(core_axis_name="c", subcore_axis_name="s")
    f = pl.kernel(
        _pool_body,
        out_type=jax.ShapeDtypeStruct((BATCH, EMBED_DIM), jnp.float32),
        mesh=mesh,
        scratch_types=[
            pltpu.VMEM((B_PER_W, SEQ), jnp.int32),
            pltpu.VMEM((NBUF, SEQ, ROW), jnp.float32),
            pltpu.VMEM((B_PER_W, EMBED_DIM), jnp.float32),
            pltpu.SemaphoreType.DMA((NBUF,)),
        ],
        compiler_params=pltpu.CompilerParams(use_tc_tiling_on_sc=False),
    )
    return f(input_ids, table128)


BM = 256


def _matmul_body(p_ref, w_ref, b_ref, o_ref):
    acc = lax.dot_general(
        p_ref[...], w_ref[...],
        dimension_numbers=(((1,), (1,)), ((), ())),
        preferred_element_type=jnp.float32)
    o_ref[...] = acc + b_ref[...]


def _tc_head(pooled, fc_w, fc_b):
    bias = fc_b.reshape(1, NUM_CLASS)
    return pl.pallas_call(
        _matmul_body,
        grid=(BATCH // BM,),
        in_specs=[
            pl.BlockSpec((BM, EMBED_DIM), lambda i: (i, 0)),
            pl.BlockSpec((NUM_CLASS, EMBED_DIM), lambda i: (0, 0)),
            pl.BlockSpec((1, NUM_CLASS), lambda i: (0, 0)),
        ],
        out_specs=pl.BlockSpec((BM, NUM_CLASS), lambda i: (i, 0)),
        out_shape=jax.ShapeDtypeStruct((BATCH, NUM_CLASS), jnp.float32),
    )(pooled, fc_w, bias)


def kernel(input_ids, emb_table, fc_w, fc_b):
    table128 = jnp.pad(emb_table, ((0, 0), (0, ROW - EMBED_DIM)))
    pooled = _sc_pool(input_ids, table128)
    return _tc_head(pooled, fc_w, fc_b)
